# baseline (speedup 1.0000x reference)
import functools

import jax
import jax.numpy as jnp
from jax.experimental import pallas as pl
from jax.experimental.pallas import tpu as pltpu

_S = 64.0
_M = 0.4

_BATCH_BLK = 512
_COL_BLK = 2048


def _margin_scale_body(lbl_ref, cos_ref, out_ref):
    j = pl.program_id(1)
    col0 = j * _COL_BLK
    lbl = lbl_ref[...]
    cols = jax.lax.broadcasted_iota(jnp.int32, cos_ref.shape, 1) + col0
    hit = lbl == cols
    out_ref[...] = cos_ref[...] * _S - jnp.where(hit, _M * _S, 0.0).astype(
        cos_ref.dtype
    )


def kernel(cosine, label):
    batch, num_classes = cosine.shape
    lbl2d = label.astype(jnp.int32).reshape(batch, 1)
    grid = (batch // _BATCH_BLK, pl.cdiv(num_classes, _COL_BLK))
    return pl.pallas_call(
        _margin_scale_body,
        grid=grid,
        in_specs=[
            pl.BlockSpec((_BATCH_BLK, 1), lambda i, j: (i, 0)),
            pl.BlockSpec((_BATCH_BLK, _COL_BLK), lambda i, j: (i, j)),
        ],
        out_specs=pl.BlockSpec((_BATCH_BLK, _COL_BLK), lambda i, j: (i, j)),
        out_shape=jax.ShapeDtypeStruct((batch, num_classes), cosine.dtype),
    )(lbl2d, cosine)


# speedup vs baseline: 1.0000x; 1.0000x over previous
import functools

import jax
import jax.numpy as jnp
from jax.experimental import pallas as pl
from jax.experimental.pallas import tpu as pltpu

_S = 64.0
_M = 0.4

_BATCH_BLK = 16
_COL_BLK = 100000


def _margin_scale_body(lbl_ref, cos_ref, out_ref):
    j = pl.program_id(1)
    col0 = j * _COL_BLK
    lbl = lbl_ref[...]
    cols = jax.lax.broadcasted_iota(jnp.int32, cos_ref.shape, 1) + col0
    hit = lbl == cols
    out_ref[...] = cos_ref[...] * _S - jnp.where(hit, _M * _S, 0.0).astype(
        cos_ref.dtype
    )


def kernel(cosine, label):
    batch, num_classes = cosine.shape
    lbl2d = label.astype(jnp.int32).reshape(batch, 1)
    grid = (batch // _BATCH_BLK, pl.cdiv(num_classes, _COL_BLK))
    return pl.pallas_call(
        _margin_scale_body,
        grid=grid,
        in_specs=[
            pl.BlockSpec((_BATCH_BLK, 1), lambda i, j: (i, 0)),
            pl.BlockSpec((_BATCH_BLK, _COL_BLK), lambda i, j: (i, j)),
        ],
        out_specs=pl.BlockSpec((_BATCH_BLK, _COL_BLK), lambda i, j: (i, j)),
        out_shape=jax.ShapeDtypeStruct((batch, num_classes), cosine.dtype),
    )(lbl2d, cosine)


# speedup vs baseline: 1.0019x; 1.0019x over previous
import functools

import jax
import jax.numpy as jnp
from jax.experimental import pallas as pl
from jax.experimental.pallas import tpu as pltpu

_S = 64.0
_M = 0.4

_R = 8
_NBUF = 6


def _body(lbl_ref, cos_hbm, out_hbm, inbufs, outbufs, insems, outsems):
    i = pl.program_id(0)
    nchunk = pl.num_programs(0)
    slot = jax.lax.rem(i, _NBUF)

    @pl.when(i == 0)
    def _prologue():
        for k in range(_NBUF):
            pltpu.make_async_copy(
                cos_hbm.at[pl.ds(k * _R, _R)], inbufs.at[k], insems.at[k]
            ).start()

    pltpu.make_async_copy(
        cos_hbm.at[pl.ds(i * _R, _R)], inbufs.at[slot], insems.at[slot]
    ).wait()

    @pl.when(i >= _NBUF)
    def _drain_prev():
        pltpu.make_async_copy(
            outbufs.at[slot],
            out_hbm.at[pl.ds((i - _NBUF) * _R, _R)],
            outsems.at[slot],
        ).wait()

    lbl = lbl_ref[pl.ds(i * _R, _R)]
    cols = jax.lax.broadcasted_iota(jnp.int32, (_R, cos_hbm.shape[1]), 1)
    hit = lbl == cols
    outbufs[slot] = inbufs[slot] * _S - jnp.where(hit, _M * _S, 0.0)

    pltpu.make_async_copy(
        outbufs.at[slot], out_hbm.at[pl.ds(i * _R, _R)], outsems.at[slot]
    ).start()

    @pl.when(i + _NBUF < nchunk)
    def _next_in():
        pltpu.make_async_copy(
            cos_hbm.at[pl.ds((i + _NBUF) * _R, _R)], inbufs.at[slot], insems.at[slot]
        ).start()

    @pl.when(i == nchunk - 1)
    def _epilogue():
        for j in range(_NBUF):
            s = nchunk - _NBUF + j
            pltpu.make_async_copy(
                outbufs.at[s % _NBUF],
                out_hbm.at[pl.ds(s * _R, _R)],
                outsems.at[s % _NBUF],
            ).wait()


def kernel(cosine, label):
    batch, num_classes = cosine.shape
    lbl2d = label.astype(jnp.int32).reshape(batch, 1)
    nchunk = batch // _R
    return pl.pallas_call(
        _body,
        grid=(nchunk,),
        in_specs=[
            pl.BlockSpec(memory_space=pltpu.VMEM),
            pl.BlockSpec(memory_space=pl.ANY),
        ],
        out_specs=pl.BlockSpec(memory_space=pl.ANY),
        out_shape=jax.ShapeDtypeStruct((batch, num_classes), cosine.dtype),
        scratch_shapes=[
            pltpu.VMEM((_NBUF, _R, num_classes), cosine.dtype),
            pltpu.VMEM((_NBUF, _R, num_classes), cosine.dtype),
            pltpu.SemaphoreType.DMA((_NBUF,)),
            pltpu.SemaphoreType.DMA((_NBUF,)),
        ],
    )(lbl2d, cosine)


# speedup vs baseline: 3.6903x; 3.6831x over previous
import jax
import jax.numpy as jnp
from jax.experimental import pallas as pl
from jax.experimental.pallas import tpu as pltpu

_S = 64.0
_M = 0.4

_ROW_BLK = 800


def _body(lbl_ref, cos_ref, out_ref):
    i = pl.program_id(0)
    classes = (
        jax.lax.broadcasted_iota(jnp.int32, out_ref.shape, 0) + i * _ROW_BLK
    )
    hit = lbl_ref[...] == classes
    out_ref[...] = cos_ref[...] * _S - jnp.where(hit, _M * _S, 0.0).astype(
        cos_ref.dtype
    )


def kernel(cosine, label):
    batch, num_classes = cosine.shape
    cos_t = cosine.T
    lbl2d = label.astype(jnp.int32).reshape(1, batch)
    out_t = pl.pallas_call(
        _body,
        grid=(num_classes // _ROW_BLK,),
        in_specs=[
            pl.BlockSpec((1, batch), lambda i: (0, 0)),
            pl.BlockSpec((_ROW_BLK, batch), lambda i: (i, 0)),
        ],
        out_specs=pl.BlockSpec((_ROW_BLK, batch), lambda i: (i, 0)),
        out_shape=jax.ShapeDtypeStruct((num_classes, batch), cosine.dtype),
    )(lbl2d, cos_t)
    return out_t.T


# speedup vs baseline: 3.8054x; 1.0312x over previous
import jax
import jax.numpy as jnp
from jax.experimental import pallas as pl
from jax.experimental.pallas import tpu as pltpu

_S = 64.0
_M = 0.4

_ROW_BLK = 2000


def _body(lbl_ref, cos_ref, out_ref):
    i = pl.program_id(0)
    classes = (
        jax.lax.broadcasted_iota(jnp.int32, out_ref.shape, 0) + i * _ROW_BLK
    )
    hit = lbl_ref[...] == classes
    out_ref[...] = cos_ref[...] * _S - jnp.where(hit, _M * _S, 0.0).astype(
        cos_ref.dtype
    )


def kernel(cosine, label):
    batch, num_classes = cosine.shape
    cos_t = cosine.T
    lbl2d = label.astype(jnp.int32).reshape(1, batch)
    out_t = pl.pallas_call(
        _body,
        grid=(num_classes // _ROW_BLK,),
        in_specs=[
            pl.BlockSpec((1, batch), lambda i: (0, 0)),
            pl.BlockSpec((_ROW_BLK, batch), lambda i: (i, 0)),
        ],
        out_specs=pl.BlockSpec((_ROW_BLK, batch), lambda i: (i, 0)),
        out_shape=jax.ShapeDtypeStruct((num_classes, batch), cosine.dtype),
    )(lbl2d, cos_t)
    return out_t.T


# speedup vs baseline: 3.8236x; 1.0048x over previous
import jax
import jax.numpy as jnp
from jax.experimental import pallas as pl
from jax.experimental.pallas import tpu as pltpu

_S = 64.0
_M = 0.4

_R = 400
_NBUF = 8


def _body(lbl_ref, cos_hbm, out_hbm, inbufs, outbufs, insems, outsems):
    i = pl.program_id(0)
    nchunk = pl.num_programs(0)
    slot = jax.lax.rem(i, _NBUF)

    @pl.when(i == 0)
    def _prologue():
        for k in range(_NBUF):
            pltpu.make_async_copy(
                cos_hbm.at[pl.ds(k * _R, _R)], inbufs.at[k], insems.at[k]
            ).start()

    pltpu.make_async_copy(
        cos_hbm.at[pl.ds(i * _R, _R)], inbufs.at[slot], insems.at[slot]
    ).wait()

    @pl.when(i >= _NBUF)
    def _drain_prev():
        pltpu.make_async_copy(
            outbufs.at[slot],
            out_hbm.at[pl.ds((i - _NBUF) * _R, _R)],
            outsems.at[slot],
        ).wait()

    classes = jax.lax.broadcasted_iota(jnp.int32, (_R, cos_hbm.shape[1]), 0) + i * _R
    hit = lbl_ref[...] == classes
    outbufs[slot] = inbufs[slot] * _S - jnp.where(hit, _M * _S, 0.0)

    pltpu.make_async_copy(
        outbufs.at[slot], out_hbm.at[pl.ds(i * _R, _R)], outsems.at[slot]
    ).start()

    @pl.when(i + _NBUF < nchunk)
    def _next_in():
        pltpu.make_async_copy(
            cos_hbm.at[pl.ds((i + _NBUF) * _R, _R)], inbufs.at[slot], insems.at[slot]
        ).start()

    @pl.when(i == nchunk - 1)
    def _epilogue():
        for j in range(_NBUF):
            s = nchunk - _NBUF + j
            pltpu.make_async_copy(
                outbufs.at[s % _NBUF],
                out_hbm.at[pl.ds(s * _R, _R)],
                outsems.at[s % _NBUF],
            ).wait()


def kernel(cosine, label):
    batch, num_classes = cosine.shape
    cos_t = cosine.T
    lbl2d = label.astype(jnp.int32).reshape(1, batch)
    nchunk = num_classes // _R
    out_t = pl.pallas_call(
        _body,
        grid=(nchunk,),
        in_specs=[
            pl.BlockSpec(memory_space=pltpu.VMEM),
            pl.BlockSpec(memory_space=pl.ANY),
        ],
        out_specs=pl.BlockSpec(memory_space=pl.ANY),
        out_shape=jax.ShapeDtypeStruct((num_classes, batch), cosine.dtype),
        scratch_shapes=[
            pltpu.VMEM((_NBUF, _R, batch), cosine.dtype),
            pltpu.VMEM((_NBUF, _R, batch), cosine.dtype),
            pltpu.SemaphoreType.DMA((_NBUF,)),
            pltpu.SemaphoreType.DMA((_NBUF,)),
        ],
    )(lbl2d, cos_t)
    return out_t.T
